# baseline (speedup 1.0000x reference)
import functools

import jax
import jax.numpy as jnp
from jax import lax
from jax.experimental import pallas as pl
from jax.experimental.pallas import tpu as pltpu
from jax.experimental.pallas import tpu_sc as plsc

N_TOK = 16384
K_TOT = 8192
E_DIM = 32
SHARE = 4096
P1 = 6144
ROW_TILE = 256
N_ROW_TILES = N_TOK // ROW_TILE
MASK_VAL = 1e7
BETA_C = 0.25

SC_NC = 2
SC_NS = 16
SC_NW = SC_NC * SC_NS
ROWS_PER_W = N_TOK // SC_NW
IDX_CHUNK = 128
N_CHUNKS = ROWS_PER_W // IDX_CHUNK


K_ACT = P1
SPEC = K_TOT - P1


def _dist_body(x_ref, et_ref, d_ref, idx_ref):
    i = pl.program_id(0)
    half = i // (N_ROW_TILES // 2)
    xt = x_ref[...]
    et = et_ref[0]
    x2 = jnp.sum(xt * xt, axis=1, keepdims=True)
    e2 = jnp.sum(et * et, axis=0, keepdims=True)
    cross = jnp.dot(xt, et)
    d = x2 + e2 - 2.0 * cross
    d_ref[:, :SHARE] = d[:, :SHARE]
    off_act = SHARE + SPEC * half
    off_msk = P1 - SPEC * half
    d_ref[:, pl.ds(off_act, SPEC)] = d[:, SHARE:]
    d_ref[:, pl.ds(off_msk, SPEC)] = jnp.full(
        (ROW_TILE, SPEC), MASK_VAL, jnp.float32)
    idxp = jnp.argmin(d, axis=1).astype(jnp.int32)
    shift = (SPEC * half).astype(jnp.int32)
    idx = idxp + jnp.where(idxp >= SHARE, shift, 0)
    idx_ref[0, 0, :] = idx


_dist_call = pl.pallas_call(
    _dist_body,
    grid=(N_ROW_TILES,),
    in_specs=[
        pl.BlockSpec((ROW_TILE, E_DIM), lambda i: (i, 0)),
        pl.BlockSpec((1, E_DIM, K_ACT), lambda i: (i // (N_ROW_TILES // 2), 0, 0)),
    ],
    out_specs=[
        pl.BlockSpec((ROW_TILE, K_TOT), lambda i: (i, 0)),
        pl.BlockSpec((1, 1, ROW_TILE), lambda i: (i, 0, 0)),
    ],
    out_shape=[
        jax.ShapeDtypeStruct((N_TOK, K_TOT), jnp.float32),
        jax.ShapeDtypeStruct((N_ROW_TILES, 1, ROW_TILE), jnp.int32),
    ],
)


@functools.lru_cache(maxsize=None)
def _make_gather():
    @functools.partial(
        pl.kernel,
        out_type=jax.ShapeDtypeStruct((N_TOK, E_DIM), jnp.float32),
        mesh=plsc.VectorSubcoreMesh(core_axis_name="c", subcore_axis_name="s"),
        compiler_params=pltpu.CompilerParams(use_tc_tiling_on_sc=False),
        scratch_types=[
            pltpu.VMEM((N_CHUNKS, IDX_CHUNK), jnp.int32),
            pltpu.VMEM((ROWS_PER_W, E_DIM), jnp.float32),
            pltpu.SemaphoreType.DMA,
        ],
    )
    def _gather_kernel(idx_hbm, table_hbm, out_hbm, idx_v, rows_v, sem):
        wid = lax.axis_index("s") * SC_NC + lax.axis_index("c")
        pltpu.sync_copy(idx_hbm.at[pl.ds(wid * N_CHUNKS, N_CHUNKS)], idx_v)
        copies = []
        for j in range(N_CHUNKS):
            copies.append(
                pltpu.async_copy(
                    table_hbm.at[idx_v.at[j]],
                    rows_v.at[pl.ds(j * IDX_CHUNK, IDX_CHUNK)],
                    sem,
                )
            )
        for c in copies:
            c.wait()
        pltpu.sync_copy(rows_v, out_hbm.at[pl.ds(wid * ROWS_PER_W, ROWS_PER_W)])

    return _gather_kernel


def _finish_body(x_ref, xq_ref, st_ref, loss_ref):
    xt = x_ref[...]
    xq = xq_ref[...]
    diff = xq - xt
    st_ref[...] = xt + diff
    sq = diff * diff
    half = N_TOK // 2
    m0 = jnp.mean(sq[:half])
    m1 = jnp.mean(sq[half:])
    loss_ref[0] = m0 + BETA_C * m0
    loss_ref[1] = m1 + BETA_C * m1


_finish_call = pl.pallas_call(
    _finish_body,
    out_specs=[
        pl.BlockSpec(memory_space=pltpu.VMEM),
        pl.BlockSpec(memory_space=pltpu.SMEM),
    ],
    out_shape=[
        jax.ShapeDtypeStruct((N_TOK, E_DIM), jnp.float32),
        jax.ShapeDtypeStruct((2,), jnp.float32),
    ],
)


def kernel(x, emb_weight, split_index):
    del split_index
    emb_t = emb_weight.T
    et_stack = jnp.stack([
        emb_t[:, :P1],
        jnp.concatenate([emb_t[:, :SHARE], emb_t[:, P1:]], axis=1),
    ])
    d, idx3 = _dist_call(x, et_stack)
    indices = idx3.reshape(N_TOK)
    x_q = _make_gather()(idx3.reshape(N_TOK // IDX_CHUNK, IDX_CHUNK), emb_weight)
    x_q_st, q_losses = _finish_call(x, x_q)
    return (x_q_st, indices, d, q_losses)


# speedup vs baseline: 1.0298x; 1.0298x over previous
import functools

import jax
import jax.numpy as jnp
from jax import lax
from jax.experimental import pallas as pl
from jax.experimental.pallas import tpu as pltpu
from jax.experimental.pallas import tpu_sc as plsc

N_TOK = 16384
K_TOT = 8192
E_DIM = 32
SHARE = 4096
P1 = 6144
ROW_TILE = 512
N_ROW_TILES = N_TOK // ROW_TILE
MASK_VAL = 1e7
BETA_C = 0.25

SC_NC = 2
SC_NS = 16
SC_NW = SC_NC * SC_NS
ROWS_PER_W = N_TOK // SC_NW
IDX_CHUNK = 128
N_CHUNKS = ROWS_PER_W // IDX_CHUNK


K_ACT = P1
SPEC = K_TOT - P1


def _dist_body(x_ref, et_ref, d_ref, idx_ref):
    i = pl.program_id(0)
    half = i // (N_ROW_TILES // 2)
    xt = x_ref[...]
    et = et_ref[0]
    x2 = jnp.sum(xt * xt, axis=1, keepdims=True)
    e2 = jnp.sum(et * et, axis=0, keepdims=True)
    cross = jnp.dot(xt, et)
    d = x2 + e2 - 2.0 * cross
    d_ref[:, :SHARE] = d[:, :SHARE]
    off_act = SHARE + SPEC * half
    off_msk = P1 - SPEC * half
    d_ref[:, pl.ds(off_act, SPEC)] = d[:, SHARE:]
    d_ref[:, pl.ds(off_msk, SPEC)] = jnp.full(
        (ROW_TILE, SPEC), MASK_VAL, jnp.float32)
    idxp = jnp.argmin(d, axis=1).astype(jnp.int32)
    shift = (SPEC * half).astype(jnp.int32)
    idx = idxp + jnp.where(idxp >= SHARE, shift, 0)
    idx_ref[0, 0, :] = idx


_dist_call = pl.pallas_call(
    _dist_body,
    grid=(N_ROW_TILES,),
    in_specs=[
        pl.BlockSpec((ROW_TILE, E_DIM), lambda i: (i, 0)),
        pl.BlockSpec((1, E_DIM, K_ACT), lambda i: (i // (N_ROW_TILES // 2), 0, 0)),
    ],
    out_specs=[
        pl.BlockSpec((ROW_TILE, K_TOT), lambda i: (i, 0)),
        pl.BlockSpec((1, 1, ROW_TILE), lambda i: (i, 0, 0)),
    ],
    out_shape=[
        jax.ShapeDtypeStruct((N_TOK, K_TOT), jnp.float32),
        jax.ShapeDtypeStruct((N_ROW_TILES, 1, ROW_TILE), jnp.int32),
    ],
)


@functools.lru_cache(maxsize=None)
def _make_gather():
    @functools.partial(
        pl.kernel,
        out_type=jax.ShapeDtypeStruct((N_TOK, E_DIM), jnp.float32),
        mesh=plsc.VectorSubcoreMesh(core_axis_name="c", subcore_axis_name="s"),
        compiler_params=pltpu.CompilerParams(use_tc_tiling_on_sc=False),
        scratch_types=[
            pltpu.VMEM((N_CHUNKS, IDX_CHUNK), jnp.int32),
            pltpu.VMEM((ROWS_PER_W, E_DIM), jnp.float32),
            pltpu.SemaphoreType.DMA,
        ],
    )
    def _gather_kernel(idx_hbm, table_hbm, out_hbm, idx_v, rows_v, sem):
        wid = lax.axis_index("s") * SC_NC + lax.axis_index("c")
        pltpu.sync_copy(idx_hbm.at[pl.ds(wid * N_CHUNKS, N_CHUNKS)], idx_v)
        copies = []
        for j in range(N_CHUNKS):
            copies.append(
                pltpu.async_copy(
                    table_hbm.at[idx_v.at[j]],
                    rows_v.at[pl.ds(j * IDX_CHUNK, IDX_CHUNK)],
                    sem,
                )
            )
        for c in copies:
            c.wait()
        pltpu.sync_copy(rows_v, out_hbm.at[pl.ds(wid * ROWS_PER_W, ROWS_PER_W)])

    return _gather_kernel


def _finish_body(x_ref, xq_ref, st_ref, loss_ref):
    xt = x_ref[...]
    xq = xq_ref[...]
    diff = xq - xt
    st_ref[...] = xt + diff
    sq = diff * diff
    half = N_TOK // 2
    m0 = jnp.mean(sq[:half])
    m1 = jnp.mean(sq[half:])
    loss_ref[0] = m0 + BETA_C * m0
    loss_ref[1] = m1 + BETA_C * m1


_finish_call = pl.pallas_call(
    _finish_body,
    out_specs=[
        pl.BlockSpec(memory_space=pltpu.VMEM),
        pl.BlockSpec(memory_space=pltpu.SMEM),
    ],
    out_shape=[
        jax.ShapeDtypeStruct((N_TOK, E_DIM), jnp.float32),
        jax.ShapeDtypeStruct((2,), jnp.float32),
    ],
)


def kernel(x, emb_weight, split_index):
    del split_index
    emb_t = emb_weight.T
    et_stack = jnp.stack([
        emb_t[:, :P1],
        jnp.concatenate([emb_t[:, :SHARE], emb_t[:, P1:]], axis=1),
    ])
    d, idx3 = _dist_call(x, et_stack)
    indices = idx3.reshape(N_TOK)
    x_q = _make_gather()(idx3.reshape(N_TOK // IDX_CHUNK, IDX_CHUNK), emb_weight)
    x_q_st, q_losses = _finish_call(x, x_q)
    return (x_q_st, indices, d, q_losses)


# speedup vs baseline: 1.0485x; 1.0181x over previous
import functools

import jax
import jax.numpy as jnp
from jax import lax
from jax.experimental import pallas as pl
from jax.experimental.pallas import tpu as pltpu
from jax.experimental.pallas import tpu_sc as plsc

N_TOK = 16384
K_TOT = 8192
E_DIM = 32
SHARE = 4096
P1 = 6144
ROW_TILE = 512
N_ROW_TILES = N_TOK // ROW_TILE
MASK_VAL = 1e7
BETA_C = 0.25

SC_NC = 2
SC_NS = 16
SC_NW = SC_NC * SC_NS
ROWS_PER_W = N_TOK // SC_NW
IDX_CHUNK = 128
N_CHUNKS = ROWS_PER_W // IDX_CHUNK


K_ACT = P1
SPEC = K_TOT - P1


_RHS_T = (((1,), (1,)), ((), ()))


def _dist_body(x_ref, es_ref, ep_ref, d_ref, idx_ref, et_ref, e2_ref):
    i = pl.program_id(0)
    half = i // (N_ROW_TILES // 2)
    xt = x_ref[...]

    @pl.when((i == 0) | (i == N_ROW_TILES // 2))
    def _():
        et_ref[:, :SHARE] = jnp.swapaxes(es_ref[...], 0, 1)
        et_ref[:, SHARE:] = jnp.swapaxes(ep_ref[...], 0, 1)
        et = et_ref[...]
        e2_ref[...] = jnp.sum(et * et, axis=0, keepdims=True)

    x2 = jnp.sum(xt * xt, axis=1, keepdims=True)
    cross = jnp.dot(xt, et_ref[...])
    d = x2 + e2_ref[...] - 2.0 * cross
    d_ref[:, :SHARE] = d[:, :SHARE]
    off_act = SHARE + SPEC * half
    off_msk = P1 - SPEC * half
    d_ref[:, pl.ds(off_act, SPEC)] = d[:, SHARE:]
    d_ref[:, pl.ds(off_msk, SPEC)] = jnp.full(
        (ROW_TILE, SPEC), MASK_VAL, jnp.float32)
    idxp = jnp.argmin(d, axis=1).astype(jnp.int32)
    shift = (SPEC * half).astype(jnp.int32)
    idx_ref[0, 0, :] = idxp + jnp.where(idxp >= SHARE, shift, 0)


_dist_call = pl.pallas_call(
    _dist_body,
    grid=(N_ROW_TILES,),
    in_specs=[
        pl.BlockSpec((ROW_TILE, E_DIM), lambda i: (i, 0)),
        pl.BlockSpec((SHARE, E_DIM), lambda i: (0, 0)),
        pl.BlockSpec((SPEC, E_DIM), lambda i: (2 + i // (N_ROW_TILES // 2), 0)),
    ],
    out_specs=[
        pl.BlockSpec((ROW_TILE, K_TOT), lambda i: (i, 0)),
        pl.BlockSpec((1, 1, ROW_TILE), lambda i: (i, 0, 0)),
    ],
    out_shape=[
        jax.ShapeDtypeStruct((N_TOK, K_TOT), jnp.float32),
        jax.ShapeDtypeStruct((N_ROW_TILES, 1, ROW_TILE), jnp.int32),
    ],
    scratch_shapes=[
        pltpu.VMEM((E_DIM, K_ACT), jnp.float32),
        pltpu.VMEM((1, K_ACT), jnp.float32),
    ],
)


@functools.lru_cache(maxsize=None)
def _make_gather():
    @functools.partial(
        pl.kernel,
        out_type=jax.ShapeDtypeStruct((N_TOK, E_DIM), jnp.float32),
        mesh=plsc.VectorSubcoreMesh(core_axis_name="c", subcore_axis_name="s"),
        compiler_params=pltpu.CompilerParams(use_tc_tiling_on_sc=False),
        scratch_types=[
            pltpu.VMEM((N_CHUNKS, IDX_CHUNK), jnp.int32),
            pltpu.VMEM((ROWS_PER_W, E_DIM), jnp.float32),
            pltpu.SemaphoreType.DMA,
        ],
    )
    def _gather_kernel(idx_hbm, table_hbm, out_hbm, idx_v, rows_v, sem):
        wid = lax.axis_index("s") * SC_NC + lax.axis_index("c")
        pltpu.sync_copy(idx_hbm.at[pl.ds(wid * N_CHUNKS, N_CHUNKS)], idx_v)
        copies = []
        for j in range(N_CHUNKS):
            copies.append(
                pltpu.async_copy(
                    table_hbm.at[idx_v.at[j]],
                    rows_v.at[pl.ds(j * IDX_CHUNK, IDX_CHUNK)],
                    sem,
                )
            )
        for c in copies:
            c.wait()
        pltpu.sync_copy(rows_v, out_hbm.at[pl.ds(wid * ROWS_PER_W, ROWS_PER_W)])

    return _gather_kernel


FIN_TILE = 2048
N_FIN_TILES = N_TOK // FIN_TILE


def _finish_body(x_ref, xq_ref, st_ref, loss_ref, acc_ref):
    t = pl.program_id(0)
    xt = x_ref[...]
    xq = xq_ref[...]
    diff = xq - xt
    st_ref[...] = xt + diff
    part = jnp.sum(diff * diff)

    @pl.when(t == 0)
    def _():
        acc_ref[0] = 0.0
        acc_ref[1] = 0.0

    h = t // (N_FIN_TILES // 2)
    acc_ref[h] = acc_ref[h] + part

    @pl.when(t == N_FIN_TILES - 1)
    def _():
        denom = jnp.float32(N_TOK // 2 * E_DIM)
        m0 = acc_ref[0] / denom
        m1 = acc_ref[1] / denom
        loss_ref[0] = m0 + BETA_C * m0
        loss_ref[1] = m1 + BETA_C * m1


_finish_call = pl.pallas_call(
    _finish_body,
    grid=(N_FIN_TILES,),
    in_specs=[
        pl.BlockSpec((FIN_TILE, E_DIM), lambda t: (t, 0)),
        pl.BlockSpec((FIN_TILE, E_DIM), lambda t: (t, 0)),
    ],
    out_specs=[
        pl.BlockSpec((FIN_TILE, E_DIM), lambda t: (t, 0)),
        pl.BlockSpec(memory_space=pltpu.SMEM),
    ],
    out_shape=[
        jax.ShapeDtypeStruct((N_TOK, E_DIM), jnp.float32),
        jax.ShapeDtypeStruct((2,), jnp.float32),
    ],
    scratch_shapes=[pltpu.SMEM((2,), jnp.float32)],
)


def kernel(x, emb_weight, split_index):
    del split_index
    d, idx3 = _dist_call(x, emb_weight, emb_weight)
    indices = idx3.reshape(N_TOK)
    x_q = _make_gather()(idx3.reshape(N_TOK // IDX_CHUNK, IDX_CHUNK), emb_weight)
    x_q_st, q_losses = _finish_call(x, x_q)
    return (x_q_st, indices, d, q_losses)


# speedup vs baseline: 1.0535x; 1.0048x over previous
import functools

import jax
import jax.numpy as jnp
from jax import lax
from jax.experimental import pallas as pl
from jax.experimental.pallas import tpu as pltpu
from jax.experimental.pallas import tpu_sc as plsc

N_TOK = 16384
K_TOT = 8192
E_DIM = 32
SHARE = 4096
P1 = 6144
ROW_TILE = 512
N_ROW_TILES = N_TOK // ROW_TILE
MASK_VAL = 1e7
BETA_C = 0.25

SC_NC = 2
SC_NS = 16
SC_NW = SC_NC * SC_NS
ROWS_PER_W = N_TOK // SC_NW
IDX_CHUNK = 128
N_CHUNKS = ROWS_PER_W // IDX_CHUNK


K_ACT = P1
SPEC = K_TOT - P1


_RHS_T = (((1,), (1,)), ((), ()))


def _dist_body(x_ref, es_ref, ep_ref, d_ref, idx_ref, et_ref, e2_ref):
    i = pl.program_id(0)
    half = i // (N_ROW_TILES // 2)
    xt = x_ref[...]

    @pl.when((i == 0) | (i == N_ROW_TILES // 2))
    def _():
        et_ref[:, :SHARE] = jnp.swapaxes(es_ref[...], 0, 1)
        et_ref[:, SHARE:] = jnp.swapaxes(ep_ref[...], 0, 1)
        et = et_ref[...]
        e2_ref[...] = jnp.sum(et * et, axis=0, keepdims=True)

    x2 = jnp.sum(xt * xt, axis=1, keepdims=True)
    cross = jnp.dot(xt, et_ref[...])
    d = x2 + e2_ref[...] - 2.0 * cross
    d_ref[:, :SHARE] = d[:, :SHARE]
    off_act = SHARE + SPEC * half
    off_msk = P1 - SPEC * half
    d_ref[:, pl.ds(off_act, SPEC)] = d[:, SHARE:]
    d_ref[:, pl.ds(off_msk, SPEC)] = jnp.full(
        (ROW_TILE, SPEC), MASK_VAL, jnp.float32)
    idxp = jnp.argmin(d, axis=1).astype(jnp.int32)
    shift = (SPEC * half).astype(jnp.int32)
    idx_ref[0, 0, :] = idxp + jnp.where(idxp >= SHARE, shift, 0)


_dist_call = pl.pallas_call(
    _dist_body,
    grid=(N_ROW_TILES,),
    in_specs=[
        pl.BlockSpec((ROW_TILE, E_DIM), lambda i: (i, 0)),
        pl.BlockSpec((SHARE, E_DIM), lambda i: (0, 0)),
        pl.BlockSpec((SPEC, E_DIM), lambda i: (2 + i // (N_ROW_TILES // 2), 0)),
    ],
    out_specs=[
        pl.BlockSpec((ROW_TILE, K_TOT), lambda i: (i, 0)),
        pl.BlockSpec((1, 1, ROW_TILE), lambda i: (i, 0, 0)),
    ],
    out_shape=[
        jax.ShapeDtypeStruct((N_TOK, K_TOT), jnp.float32),
        jax.ShapeDtypeStruct((N_ROW_TILES, 1, ROW_TILE), jnp.int32),
    ],
    scratch_shapes=[
        pltpu.VMEM((E_DIM, K_ACT), jnp.float32),
        pltpu.VMEM((1, K_ACT), jnp.float32),
    ],
)


@functools.lru_cache(maxsize=None)
def _make_gather():
    @functools.partial(
        pl.kernel,
        out_type=jax.ShapeDtypeStruct((N_TOK, E_DIM), jnp.float32),
        mesh=plsc.VectorSubcoreMesh(core_axis_name="c", subcore_axis_name="s"),
        compiler_params=pltpu.CompilerParams(use_tc_tiling_on_sc=False),
        scratch_types=[
            pltpu.VMEM((N_CHUNKS, IDX_CHUNK), jnp.int32),
            pltpu.VMEM((ROWS_PER_W, E_DIM), jnp.float32),
            pltpu.SemaphoreType.DMA,
        ],
    )
    def _gather_kernel(idx_hbm, table_hbm, out_hbm, idx_v, rows_v, sem):
        wid = lax.axis_index("s") * SC_NC + lax.axis_index("c")
        pltpu.sync_copy(idx_hbm.at[pl.ds(wid * N_CHUNKS, N_CHUNKS)], idx_v)
        copies = []
        for j in range(N_CHUNKS):
            copies.append(
                pltpu.async_copy(
                    table_hbm.at[idx_v.at[j]],
                    rows_v.at[pl.ds(j * IDX_CHUNK, IDX_CHUNK)],
                    sem,
                )
            )
        for c in copies:
            c.wait()
        pltpu.sync_copy(rows_v, out_hbm.at[pl.ds(wid * ROWS_PER_W, ROWS_PER_W)])

    return _gather_kernel


FIN_TILE = 512
N_FIN_TILES = N_TOK // 4 // FIN_TILE


def _finish_body(x_ref, xq_ref, st_ref, loss_ref, acc_ref):
    t = pl.program_id(0)
    xt = x_ref[...]
    xq = xq_ref[...]
    diff = xq - xt
    st_ref[...] = xt + diff
    part = jnp.sum(diff * diff)

    @pl.when(t == 0)
    def _():
        acc_ref[0] = 0.0
        acc_ref[1] = 0.0

    h = t // (N_FIN_TILES // 2)
    acc_ref[h] = acc_ref[h] + part

    @pl.when(t == N_FIN_TILES - 1)
    def _():
        denom = jnp.float32(N_TOK // 2 * E_DIM)
        m0 = acc_ref[0] / denom
        m1 = acc_ref[1] / denom
        loss_ref[0] = m0 + BETA_C * m0
        loss_ref[1] = m1 + BETA_C * m1


_finish_call = pl.pallas_call(
    _finish_body,
    grid=(N_FIN_TILES,),
    in_specs=[
        pl.BlockSpec((FIN_TILE, 128), lambda t: (t, 0)),
        pl.BlockSpec((FIN_TILE, 128), lambda t: (t, 0)),
    ],
    out_specs=[
        pl.BlockSpec((FIN_TILE, 128), lambda t: (t, 0)),
        pl.BlockSpec(memory_space=pltpu.SMEM),
    ],
    out_shape=[
        jax.ShapeDtypeStruct((N_TOK // 4, 128), jnp.float32),
        jax.ShapeDtypeStruct((2,), jnp.float32),
    ],
    scratch_shapes=[pltpu.SMEM((2,), jnp.float32)],
)


def kernel(x, emb_weight, split_index):
    del split_index
    d, idx3 = _dist_call(x, emb_weight, emb_weight)
    x4 = x.reshape(N_TOK // 4, 128)
    indices = idx3.reshape(N_TOK)
    x_q = _make_gather()(idx3.reshape(N_TOK // IDX_CHUNK, IDX_CHUNK), emb_weight)
    st4, q_losses = _finish_call(x4, x_q.reshape(N_TOK // 4, 128))
    return (st4.reshape(N_TOK, E_DIM), indices, d, q_losses)
